# baseline (speedup 1.0000x reference)
import functools

import jax
import jax.numpy as jnp
from jax import lax
from jax.experimental import pallas as pl
from jax.experimental.pallas import tpu as pltpu
from jax.experimental.pallas import tpu_sc as plsc

NC = 2
NS = 16
NW = NC * NS

NJ = 26
NI = 16384
NTI = NI // 128
NBLK = NJ * NTI
BPW = NBLK // NW
GB = 8
NG = BPW // GB
D = 32
B = NI * NJ

mesh = plsc.VectorSubcoreMesh(core_axis_name="c", subcore_axis_name="s")
_sc_params = pltpu.CompilerParams(use_tc_tiling_on_sc=False)


@functools.partial(
    pl.kernel,
    mesh=mesh,
    out_type=jax.ShapeDtypeStruct((B, D), jnp.float32),
    scratch_types=[
        pltpu.VMEM((BPW * 128,), jnp.int32),
        pltpu.VMEM((GB * 128, D), jnp.float32),
        pltpu.VMEM((GB * 128, D), jnp.float32),
        pltpu.SemaphoreType.DMA,
        pltpu.SemaphoreType.DMA,
        pltpu.SemaphoreType.DMA,
        pltpu.SemaphoreType.DMA,
    ],
    compiler_params=_sc_params,
)
def _gather(idxf, wt, inter, ixv, g0, g1, sg0, sg1, sw0, sw1):
    wid = lax.axis_index("s") * NC + lax.axis_index("c")
    base = wid * BPW * 128
    pltpu.sync_copy(idxf.at[pl.ds(base, BPW * 128)], ixv)

    def g_copy(g, gv, sem):
        return pltpu.make_async_copy(
            wt.at[ixv.at[pl.ds(g * (GB * 128), GB * 128)]], gv, sem
        )

    def w_copy(g, gv, sem):
        return pltpu.make_async_copy(
            gv, inter.at[pl.ds(base + g * (GB * 128), GB * 128)], sem
        )

    g_copy(0, g0, sg0).start()

    def body(g, _):
        @pl.when(lax.rem(g, 2) == 0)
        def _():
            @pl.when(g + 1 < NG)
            def _():
                @pl.when(g >= 1)
                def _():
                    w_copy(g - 1, g1, sw1).wait()

                g_copy(g + 1, g1, sg1).start()

            g_copy(g, g0, sg0).wait()
            w_copy(g, g0, sw0).start()

        @pl.when(lax.rem(g, 2) == 1)
        def _():
            @pl.when(g + 1 < NG)
            def _():
                w_copy(g - 1, g0, sw0).wait()
                g_copy(g + 1, g0, sg0).start()

            g_copy(g, g1, sg1).wait()
            w_copy(g, g1, sw1).start()

        return ()

    lax.fori_loop(0, NG, body, ())
    w_copy(NG - 2, g0, sw0).wait()
    w_copy(NG - 1, g1, sw1).wait()


@functools.partial(
    pl.kernel,
    mesh=mesh,
    out_type=jax.ShapeDtypeStruct((NJ, 4, NTI, 1024), jnp.float32),
    scratch_types=[
        pltpu.VMEM((4096,), jnp.float32),
        pltpu.VMEM((4096,), jnp.float32),
        pltpu.VMEM((4096,), jnp.float32),
        pltpu.VMEM((4096,), jnp.float32),
        pltpu.SemaphoreType.DMA,
        pltpu.SemaphoreType.DMA,
        pltpu.SemaphoreType.DMA,
        pltpu.SemaphoreType.DMA,
    ],
    compiler_params=pltpu.CompilerParams(
        use_tc_tiling_on_sc=False, needs_layout_passes=False
    ),
)
def _relayout(interf, out, f0, f1, t0, t1, sf0, sf1, sw0, sw1):
    wid = lax.axis_index("s") * NC + lax.axis_index("c")
    base = wid * BPW
    iota = lax.broadcasted_iota(jnp.int32, (16,), 0)

    def r_copy(bs, fv, sem):
        return pltpu.make_async_copy(
            interf.at[pl.ds((base + bs) * 4096, 4096)], fv, sem
        )

    def w_copies(bs, tv, sem):
        bid = base + bs
        j = bid // NTI
        ti = bid - j * NTI
        return [
            pltpu.make_async_copy(
                tv.at[pl.ds(q * 1024, 1024)], out.at[j, q, ti], sem
            )
            for q in range(4)
        ]

    def transpose(fv, tv):
        for c in range(D):
            for l8 in range(8):
                v = plsc.load_gather(fv, [(iota + l8 * 16) * D + c])
                tv[pl.ds(c * 128 + l8 * 16, 16)] = v

    def block(bs, fv, tv, semf, semw):
        r_copy(bs, fv, semf).wait()

        @pl.when(bs >= 2)
        def _():
            for cpy in w_copies(bs - 2, tv, semw):
                cpy.wait()

        transpose(fv, tv)
        for cpy in w_copies(bs, tv, semw):
            cpy.start()

    r_copy(0, f0, sf0).start()

    def body(k2, _):
        bs = k2 * 2

        @pl.when(bs + 1 < BPW)
        def _():
            r_copy(bs + 1, f1, sf1).start()

        block(bs, f0, t0, sf0, sw0)

        @pl.when(bs + 2 < BPW)
        def _():
            r_copy(bs + 2, f0, sf0).start()

        block(bs + 1, f1, t1, sf1, sw1)
        return ()

    lax.fori_loop(0, BPW // 2, body, ())
    for cpy in w_copies(BPW - 2, t0, sw0):
        cpy.wait()
    for cpy in w_copies(BPW - 1, t1, sw1):
        cpy.wait()


def kernel(idx, weight):
    idxf = idx.T.reshape(-1).astype(jnp.int32)
    inter = _gather(idxf, weight)
    o = _relayout(inter.reshape(-1))
    o = o.reshape(NJ, 4, NTI, 8, 128)
    return o.transpose(2, 4, 0, 1, 3).reshape(NI, NJ, D)
